# baseline (speedup 1.0000x reference)
import jax
import jax.numpy as jnp
from jax import lax
from jax.experimental import pallas as pl
from jax.experimental.pallas import tpu as pltpu
from jax.experimental.pallas import tpu_sc as plsc

N = 10000
E = 320000
D = 128
L = 4
FA = 9
FB = 3
VA = 100
VB = 5
OUT = 128
EPS = 1e-5

NC = 2
NS = 16
NT = NC * NS
EC = E // NT
CPT = D // NT
PROW = 8 * N
SLICE = CPT * N
EBLK = 1600
NEBLK = E // EBLK
CTW = D * D

_SC_PARAMS = pltpu.CompilerParams(needs_layout_passes=False)


def _deg_body(src2_hbm, deg2_hbm, srcf_v, ones_v, zb_v, deg_sp, sem):
    del sem
    cid = lax.axis_index("c")
    sid = lax.axis_index("s")
    chunk = cid * NS + sid

    def fill(i, _):
        ones_v[pl.ds(i * 16, 16)] = jnp.full((16,), 1.0, jnp.float32)
        return 0
    lax.fori_loop(0, EC // 16, fill, 0)

    def zfill(i, _):
        zb_v[pl.ds(i * 16, 16)] = jnp.zeros((16,), jnp.float32)
        return 0
    lax.fori_loop(0, 2000 // 16, zfill, 0)

    @pl.when(sid < 5)
    def _():
        pltpu.sync_copy(zb_v, deg_sp.at[pl.ds(sid * 2000, 2000)])

    pltpu.sync_copy(src2_hbm.at[chunk, 0], srcf_v)
    plsc.subcore_barrier()
    pltpu.sync_copy(ones_v, deg_sp.at[srcf_v], add=True)
    plsc.subcore_barrier()

    @pl.when(sid == 0)
    def _():
        pltpu.sync_copy(deg_sp, deg2_hbm.at[cid, 0])


_deg_call = pl.kernel(
    _deg_body,
    out_type=jax.ShapeDtypeStruct((NC, 1, N), jnp.float32),
    compiler_params=_SC_PARAMS,
    mesh=plsc.VectorSubcoreMesh(core_axis_name="c", subcore_axis_name="s"),
    scratch_types=[
        pltpu.VMEM((EC,), jnp.int32),
        pltpu.VMEM((EC,), jnp.float32),
        pltpu.VMEM((2000,), jnp.float32),
        pltpu.VMEM_SHARED((N,), jnp.float32),
        pltpu.SemaphoreType.DMA,
    ],
)


def _edge_body(xlp_hbm, pk4_hbm, dst4_hbm, norm_hbm, ctab_hbm, accp_hbm,
               xsl, asl, normv, ctabv, pkb, dstb,
               ps0, ps1, ps2, ps3, ds0, ds1, ds2, ds3):
    cid = lax.axis_index("c")
    sid = lax.axis_index("s")
    t = cid * NS + sid
    z16 = jnp.zeros((16,), jnp.float32)
    psems = [ps0, ps1, ps2, ps3]
    dsems = [ds0, ds1, ds2, ds3]

    pltpu.sync_copy(xlp_hbm.at[pl.ds(t * PROW, SLICE)], xsl)
    pltpu.sync_copy(norm_hbm, normv)
    pltpu.sync_copy(ctab_hbm.at[0, pl.ds(t * 4 * D, 4 * D)], ctabv)

    def zfill(i, _):
        asl[pl.ds(i * 16, 16)] = z16
        return 0
    lax.fori_loop(0, SLICE // 16, zfill, 0)

    for j in range(4):
        pltpu.async_copy(pk4_hbm.at[j, 0], pkb.at[j], psems[j])
        pltpu.async_copy(dst4_hbm.at[j, 0], dstb.at[j], dsems[j])

    def blk_body(t4, _):
        for j in range(4):
            b = t4 * 4 + j
            pltpu.make_async_copy(pk4_hbm.at[0, 0], pkb.at[j],
                                  psems[j]).wait()
            pltpu.make_async_copy(dst4_hbm.at[0, 0], dstb.at[j],
                                  dsems[j]).wait()

            def eb(k, _):
                sl = pl.ds(k * 16, 16)
                w = pkb[j, sl]
                dv = dstb[j, sl]
                sv = w & 0x3FFF
                bv = w >> 14
                nv = plsc.load_gather(normv, [sv])
                for c in range(CPT):
                    cn = c * N
                    xv = plsc.load_gather(xsl, [sv + cn])
                    cv = plsc.load_gather(ctabv, [bv + c * D])
                    m = nv * jnp.maximum(xv + cv, 0.0)
                    plsc.addupdate_scatter(asl, [dv + cn], m)
                return 0
            lax.fori_loop(0, EBLK // 16, eb, 0)

            @pl.when(b + 4 < NEBLK)
            def _():
                pltpu.async_copy(pk4_hbm.at[b + 4, 0], pkb.at[j], psems[j])
                pltpu.async_copy(dst4_hbm.at[b + 4, 0], dstb.at[j], dsems[j])
        return 0
    lax.fori_loop(0, NEBLK // 4, blk_body, 0)

    pltpu.sync_copy(asl, accp_hbm.at[pl.ds(t * PROW, SLICE)])


_edge_call = pl.kernel(
    _edge_body,
    out_type=jax.ShapeDtypeStruct((NT * PROW,), jnp.float32),
    compiler_params=_SC_PARAMS,
    mesh=plsc.VectorSubcoreMesh(core_axis_name="c", subcore_axis_name="s"),
    scratch_types=[
        pltpu.VMEM((SLICE,), jnp.float32),
        pltpu.VMEM((SLICE,), jnp.float32),
        pltpu.VMEM((N,), jnp.float32),
        pltpu.VMEM((4 * D,), jnp.float32),
        pltpu.VMEM((4, EBLK), jnp.int32),
        pltpu.VMEM((4, EBLK), jnp.int32),
    ] + [pltpu.SemaphoreType.DMA] * 8,
)


def _pad_cols(xl_t):
    blocks = xl_t.reshape(NT, CPT, N)
    pad = jnp.zeros((NT, 8 - CPT, N), jnp.float32)
    return jnp.concatenate([blocks, pad], axis=1)


def _prep_body(x_ref, deg2_ref, at_ref, w0_ref, b0_ref, bt_ref,
               ex0_ref, ex1_ref, ex2_ref, src_ref,
               norm_ref, invd_ref, xl0_ref, ctabs_ref, pk_ref):
    degs = deg2_ref[0, 0] + deg2_ref[1, 0] + 1.0
    norm_ref[...] = lax.rsqrt(degs)
    invd_ref[...] = 1.0 / degs

    iota = lax.broadcasted_iota(jnp.int32, (1, VA), 1)
    h = jnp.zeros((N, D), jnp.float32)
    for f in range(FA):
        oh = (x_ref[:, f][:, None] == iota).astype(jnp.float32)
        h = h + lax.dot(oh, at_ref[f], preferred_element_type=jnp.float32)
    xl0_ref[...] = _pad_cols(
        lax.dot_general(w0_ref[...], h, (((1,), (1,)), ((), ())),
                        preferred_element_type=jnp.float32)
        + b0_ref[...][:, None])

    for i in range(L):
        b0t = jnp.transpose(bt_ref[i, 0])
        b1t = jnp.transpose(bt_ref[i, 1])
        b2t = jnp.transpose(bt_ref[i, 2])
        ct_t = (b0t[:, :, None, None] + b1t[:, None, :, None]
                + b2t[:, None, None, :]).reshape(D, VB ** FB)
        ctabs_ref[i] = jnp.concatenate(
            [ct_t, jnp.zeros((D, D - VB ** FB), jnp.float32)], axis=1)

    bidx = ex0_ref[...] * (VB * VB) + ex1_ref[...] * VB + ex2_ref[...]
    pk_ref[...] = src_ref[...] + bidx * 16384


_prep_call = pl.pallas_call(
    _prep_body,
    out_shape=[
        jax.ShapeDtypeStruct((N,), jnp.float32),
        jax.ShapeDtypeStruct((N,), jnp.float32),
        jax.ShapeDtypeStruct((NT, 8, N), jnp.float32),
        jax.ShapeDtypeStruct((L, D, D), jnp.float32),
        jax.ShapeDtypeStruct((E,), jnp.int32),
    ],
)


def _unpad(accp_ref):
    return accp_ref[...][:, :CPT, :].reshape(D, N)


def _upd_body(accp_ref, xlp_ref, norm_ref, invd_ref, root_ref, g_ref, b_ref,
              wn_ref, bn_ref, out_ref):
    acc_t = _unpad(accp_ref)
    xl_t = _unpad(xlp_ref)
    h_t = (norm_ref[...][None, :] * acc_t
           + jnp.maximum(xl_t + root_ref[...][:, None], 0.0)
           * invd_ref[...][None, :])
    scale = g_ref[...] * (1.0 / jnp.sqrt(1.0 + EPS))
    h_t = h_t * scale[:, None] + b_ref[...][:, None]
    h_t = jnp.maximum(h_t, 0.0)
    out_ref[...] = _pad_cols(
        lax.dot_general(wn_ref[...], h_t, (((1,), (0,)), ((), ())),
                        preferred_element_type=jnp.float32)
        + bn_ref[...][:, None])


_upd_call = pl.pallas_call(
    _upd_body,
    out_shape=jax.ShapeDtypeStruct((NT, 8, N), jnp.float32),
)


def _fin_body(accp_ref, xlp_ref, norm_ref, invd_ref, root_ref, wo_ref, bo_ref,
              out_ref):
    acc_t = _unpad(accp_ref)
    xl_t = _unpad(xlp_ref)
    h_t = (norm_ref[...][None, :] * acc_t
           + jnp.maximum(xl_t + root_ref[...][:, None], 0.0)
           * invd_ref[...][None, :])
    hg = jnp.sum(h_t, axis=1) * (1.0 / N)
    out_ref[...] = lax.dot_general(
        wo_ref[...], hg[:, None], (((1,), (0,)), ((), ())),
        preferred_element_type=jnp.float32).reshape(1, OUT) + bo_ref[...][None, :]


_fin_call = pl.pallas_call(
    _fin_body,
    out_shape=jax.ShapeDtypeStruct((1, OUT), jnp.float32),
)


@jax.jit
def kernel(x, edge_index, ex, atom_tables, bond_tables, Ws, bs, roots,
           bn_gamma, bn_beta, W_out, b_out):
    src = edge_index[0].astype(jnp.int32)
    dst = edge_index[1].astype(jnp.int32)
    src2 = src.reshape(NT, 1, EC)
    dst4 = dst.reshape(NEBLK, 1, EBLK)
    ex = ex.astype(jnp.int32)
    ex0, ex1, ex2 = ex[:, 0], ex[:, 1], ex[:, 2]

    deg2 = _deg_call(src2)
    norm, invd, xlp, ctabs, pk = _prep_call(
        x.astype(jnp.int32), deg2, atom_tables, Ws[0], bs[0], bond_tables,
        ex0, ex1, ex2, src)
    pk4 = pk.reshape(NEBLK, 1, EBLK)
    ctabsf = ctabs.reshape(L, 1, CTW)

    for i in range(L):
        accp = _edge_call(xlp.reshape(-1), pk4, dst4, norm, ctabsf[i])
        accp = accp.reshape(NT, 8, N)
        if i < L - 1:
            xlp = _upd_call(accp, xlp, norm, invd, roots[i], bn_gamma[i],
                            bn_beta[i], Ws[i + 1], bs[i + 1])
        else:
            out = _fin_call(accp, xlp, norm, invd, roots[L - 1], W_out, b_out)
    return out


# speedup vs baseline: 2.7215x; 2.7215x over previous
import jax
import jax.numpy as jnp
from jax import lax
from jax.experimental import pallas as pl
from jax.experimental.pallas import tpu as pltpu
from jax.experimental.pallas import tpu_sc as plsc

N = 10000
E = 320000
D = 128
L = 4
FA = 9
FB = 3
VA = 100
VB = 5
OUT = 128
EPS = 1e-5

NC = 2
NS = 16
NT = NC * NS
EC = E // NT
CPT = D // NT
PROW = 8 * N
SLICE = CPT * N
EBLK = 1600
NEBLK = E // EBLK
CTW = D * D

_SC_PARAMS = pltpu.CompilerParams(needs_layout_passes=False)


def _deg_body(src2_hbm, deg2_hbm, srcf_v, ones_v, zb_v, deg_sp, sem):
    del sem
    cid = lax.axis_index("c")
    sid = lax.axis_index("s")
    chunk = cid * NS + sid

    def fill(i, _):
        ones_v[pl.ds(i * 16, 16)] = jnp.full((16,), 1.0, jnp.float32)
        return 0
    lax.fori_loop(0, EC // 16, fill, 0)

    def zfill(i, _):
        zb_v[pl.ds(i * 16, 16)] = jnp.zeros((16,), jnp.float32)
        return 0
    lax.fori_loop(0, 2000 // 16, zfill, 0)

    @pl.when(sid < 5)
    def _():
        pltpu.sync_copy(zb_v, deg_sp.at[pl.ds(sid * 2000, 2000)])

    pltpu.sync_copy(src2_hbm.at[chunk, 0], srcf_v)
    plsc.subcore_barrier()
    pltpu.sync_copy(ones_v, deg_sp.at[srcf_v], add=True)
    plsc.subcore_barrier()

    @pl.when(sid == 0)
    def _():
        pltpu.sync_copy(deg_sp, deg2_hbm.at[cid, 0])


_deg_call = pl.kernel(
    _deg_body,
    out_type=jax.ShapeDtypeStruct((NC, 1, N), jnp.float32),
    compiler_params=_SC_PARAMS,
    mesh=plsc.VectorSubcoreMesh(core_axis_name="c", subcore_axis_name="s"),
    scratch_types=[
        pltpu.VMEM((EC,), jnp.int32),
        pltpu.VMEM((EC,), jnp.float32),
        pltpu.VMEM((2000,), jnp.float32),
        pltpu.VMEM_SHARED((N,), jnp.float32),
        pltpu.SemaphoreType.DMA,
    ],
)


def _edge_body(xlp_hbm, pk4_hbm, dst4_hbm, norm_hbm, ctab_hbm, accp_hbm,
               xsl, asl, normv, ctabv, pkb, dstb,
               ps0, ps1, ps2, ps3, ds0, ds1, ds2, ds3):
    cid = lax.axis_index("c")
    sid = lax.axis_index("s")
    t = cid * NS + sid
    z16 = jnp.zeros((16,), jnp.float32)
    psems = [ps0, ps1, ps2, ps3]
    dsems = [ds0, ds1, ds2, ds3]

    pltpu.sync_copy(xlp_hbm.at[pl.ds(t * PROW, SLICE)], xsl)
    pltpu.sync_copy(norm_hbm, normv)
    pltpu.sync_copy(ctab_hbm.at[0, pl.ds(t * 4 * D, 4 * D)], ctabv)

    def zfill(i, _):
        asl[pl.ds(i * 16, 16)] = z16
        return 0
    lax.fori_loop(0, SLICE // 16, zfill, 0)

    for j in range(4):
        pltpu.async_copy(pk4_hbm.at[j, 0], pkb.at[j], psems[j])
        pltpu.async_copy(dst4_hbm.at[j, 0], dstb.at[j], dsems[j])

    def blk_body(t4, _):
        for j in range(4):
            b = t4 * 4 + j
            pltpu.make_async_copy(pk4_hbm.at[0, 0], pkb.at[j],
                                  psems[j]).wait()
            pltpu.make_async_copy(dst4_hbm.at[0, 0], dstb.at[j],
                                  dsems[j]).wait()

            @plsc.parallel_loop(0, EBLK // 16, unroll=4)
            def eb(k):
                sl = pl.ds(k * 16, 16)
                w = pkb[j, sl]
                dv = dstb[j, sl]
                sv = w & 0x3FFF
                bv = w >> 14
                nv = plsc.load_gather(normv, [sv])
                for c in range(CPT):
                    cn = c * N
                    xv = plsc.load_gather(xsl, [sv + cn])
                    cv = plsc.load_gather(ctabv, [bv + c * D])
                    m = nv * jnp.maximum(xv + cv, 0.0)
                    plsc.addupdate_scatter(asl, [dv + cn], m)

            @pl.when(b + 4 < NEBLK)
            def _():
                pltpu.async_copy(pk4_hbm.at[b + 4, 0], pkb.at[j], psems[j])
                pltpu.async_copy(dst4_hbm.at[b + 4, 0], dstb.at[j], dsems[j])
        return 0
    lax.fori_loop(0, NEBLK // 4, blk_body, 0)

    pltpu.sync_copy(asl, accp_hbm.at[pl.ds(t * PROW, SLICE)])


_edge_call = pl.kernel(
    _edge_body,
    out_type=jax.ShapeDtypeStruct((NT * PROW,), jnp.float32),
    compiler_params=_SC_PARAMS,
    mesh=plsc.VectorSubcoreMesh(core_axis_name="c", subcore_axis_name="s"),
    scratch_types=[
        pltpu.VMEM((SLICE,), jnp.float32),
        pltpu.VMEM((SLICE,), jnp.float32),
        pltpu.VMEM((N,), jnp.float32),
        pltpu.VMEM((4 * D,), jnp.float32),
        pltpu.VMEM((4, EBLK), jnp.int32),
        pltpu.VMEM((4, EBLK), jnp.int32),
    ] + [pltpu.SemaphoreType.DMA] * 8,
)


def _pad_cols(xl_t):
    blocks = xl_t.reshape(NT, CPT, N)
    pad = jnp.zeros((NT, 8 - CPT, N), jnp.float32)
    return jnp.concatenate([blocks, pad], axis=1)


def _prep_body(x_ref, deg2_ref, at_ref, w0_ref, b0_ref, bt_ref,
               ex0_ref, ex1_ref, ex2_ref, src_ref,
               norm_ref, invd_ref, xl0_ref, ctabs_ref, pk_ref):
    degs = deg2_ref[0, 0] + deg2_ref[1, 0] + 1.0
    norm_ref[...] = lax.rsqrt(degs)
    invd_ref[...] = 1.0 / degs

    iota = lax.broadcasted_iota(jnp.int32, (1, VA), 1)
    h = jnp.zeros((N, D), jnp.float32)
    for f in range(FA):
        oh = (x_ref[:, f][:, None] == iota).astype(jnp.float32)
        h = h + lax.dot(oh, at_ref[f], preferred_element_type=jnp.float32)
    xl0_ref[...] = _pad_cols(
        lax.dot_general(w0_ref[...], h, (((1,), (1,)), ((), ())),
                        preferred_element_type=jnp.float32)
        + b0_ref[...][:, None])

    for i in range(L):
        b0t = jnp.transpose(bt_ref[i, 0])
        b1t = jnp.transpose(bt_ref[i, 1])
        b2t = jnp.transpose(bt_ref[i, 2])
        ct_t = (b0t[:, :, None, None] + b1t[:, None, :, None]
                + b2t[:, None, None, :]).reshape(D, VB ** FB)
        ctabs_ref[i] = jnp.concatenate(
            [ct_t, jnp.zeros((D, D - VB ** FB), jnp.float32)], axis=1)

    bidx = ex0_ref[...] * (VB * VB) + ex1_ref[...] * VB + ex2_ref[...]
    pk_ref[...] = src_ref[...] + bidx * 16384


_prep_call = pl.pallas_call(
    _prep_body,
    out_shape=[
        jax.ShapeDtypeStruct((N,), jnp.float32),
        jax.ShapeDtypeStruct((N,), jnp.float32),
        jax.ShapeDtypeStruct((NT, 8, N), jnp.float32),
        jax.ShapeDtypeStruct((L, D, D), jnp.float32),
        jax.ShapeDtypeStruct((E,), jnp.int32),
    ],
)


def _unpad(accp_ref):
    return accp_ref[...][:, :CPT, :].reshape(D, N)


def _upd_body(accp_ref, xlp_ref, norm_ref, invd_ref, root_ref, g_ref, b_ref,
              wn_ref, bn_ref, out_ref):
    acc_t = _unpad(accp_ref)
    xl_t = _unpad(xlp_ref)
    h_t = (norm_ref[...][None, :] * acc_t
           + jnp.maximum(xl_t + root_ref[...][:, None], 0.0)
           * invd_ref[...][None, :])
    scale = g_ref[...] * (1.0 / jnp.sqrt(1.0 + EPS))
    h_t = h_t * scale[:, None] + b_ref[...][:, None]
    h_t = jnp.maximum(h_t, 0.0)
    out_ref[...] = _pad_cols(
        lax.dot_general(wn_ref[...], h_t, (((1,), (0,)), ((), ())),
                        preferred_element_type=jnp.float32)
        + bn_ref[...][:, None])


_upd_call = pl.pallas_call(
    _upd_body,
    out_shape=jax.ShapeDtypeStruct((NT, 8, N), jnp.float32),
)


def _fin_body(accp_ref, xlp_ref, norm_ref, invd_ref, root_ref, wo_ref, bo_ref,
              out_ref):
    acc_t = _unpad(accp_ref)
    xl_t = _unpad(xlp_ref)
    h_t = (norm_ref[...][None, :] * acc_t
           + jnp.maximum(xl_t + root_ref[...][:, None], 0.0)
           * invd_ref[...][None, :])
    hg = jnp.sum(h_t, axis=1) * (1.0 / N)
    out_ref[...] = lax.dot_general(
        wo_ref[...], hg[:, None], (((1,), (0,)), ((), ())),
        preferred_element_type=jnp.float32).reshape(1, OUT) + bo_ref[...][None, :]


_fin_call = pl.pallas_call(
    _fin_body,
    out_shape=jax.ShapeDtypeStruct((1, OUT), jnp.float32),
)


@jax.jit
def kernel(x, edge_index, ex, atom_tables, bond_tables, Ws, bs, roots,
           bn_gamma, bn_beta, W_out, b_out):
    src = edge_index[0].astype(jnp.int32)
    dst = edge_index[1].astype(jnp.int32)
    src2 = src.reshape(NT, 1, EC)
    dst4 = dst.reshape(NEBLK, 1, EBLK)
    ex = ex.astype(jnp.int32)
    ex0, ex1, ex2 = ex[:, 0], ex[:, 1], ex[:, 2]

    deg2 = _deg_call(src2)
    norm, invd, xlp, ctabs, pk = _prep_call(
        x.astype(jnp.int32), deg2, atom_tables, Ws[0], bs[0], bond_tables,
        ex0, ex1, ex2, src)
    pk4 = pk.reshape(NEBLK, 1, EBLK)
    ctabsf = ctabs.reshape(L, 1, CTW)

    for i in range(L):
        accp = _edge_call(xlp.reshape(-1), pk4, dst4, norm, ctabsf[i])
        accp = accp.reshape(NT, 8, N)
        if i < L - 1:
            xlp = _upd_call(accp, xlp, norm, invd, roots[i], bn_gamma[i],
                            bn_beta[i], Ws[i + 1], bs[i + 1])
        else:
            out = _fin_call(accp, xlp, norm, invd, roots[L - 1], W_out, b_out)
    return out
